# baseline (speedup 1.0000x reference)
import functools

import jax
import jax.numpy as jnp
from jax import lax
from jax.experimental import pallas as pl
from jax.experimental.pallas import tpu as pltpu
from jax.experimental.pallas import tpu_sc as plsc

NC = 2
NS = 16
L = 16
K = 128


def _sc_aggregate(N, D, E, with_counts):
  NW = NC * NS
  assert E % NW == 0
  e_per_tile = E // NW
  n_full = e_per_tile // K
  rem = e_per_tile - n_full * K
  assert rem % 8 == 0
  rpt0 = (N // NS) // 8 * 8
  rpt_last = N - (NS - 1) * rpt0

  mesh = plsc.VectorSubcoreMesh(core_axis_name="c", subcore_axis_name="s")

  assert n_full >= 4 and n_full % 2 == 0

  out_type = [jax.ShapeDtypeStruct((NC, N, D), jnp.float32)]
  scratch = (
      [pltpu.VMEM((K,), jnp.int32)] * 2 +
      [pltpu.VMEM((K,), jnp.int32)] * 2 +
      [pltpu.VMEM((K, D), jnp.float32)] * 2 +
      [pltpu.VMEM((rem if rem else 8,), jnp.int32)] * 2 +
      [pltpu.VMEM((rem if rem else 8, D), jnp.float32)] +
      [pltpu.VMEM_SHARED((N, D), jnp.float32)] +
      [pltpu.SemaphoreType.DMA] * 2 +
      [pltpu.SemaphoreType.DMA] * 2 +
      [pltpu.SemaphoreType.DMA]
  )
  if with_counts:
    out_type.append(jax.ShapeDtypeStruct((NW * N,), jnp.float32))
    scratch.append(pltpu.VMEM((N,), jnp.float32))

  def body(x_hbm, edges_hbm, zrow_hbm, zcnt_hbm, *rest):
    if with_counts:
      out_p, out_cnt = rest[0], rest[1]
      rest = rest[2:]
    else:
      out_p = rest[0]
      rest = rest[1:]
    srcs, dsts, rows = rest[0:2], rest[2:4], rest[4:6]
    srcv_t, dstv_t, rows_t, accum = rest[6:10]
    semi, semg = rest[10:12], rest[12:14]
    sem_t = rest[14]
    cntv = rest[15] if with_counts else None

    cid = lax.axis_index("c")
    sid = lax.axis_index("s")
    wid = sid * NC + cid

    r0 = sid * rpt0
    is_last = sid == NS - 1

    @pl.when(is_last)
    def _():
      pltpu.sync_copy(zrow_hbm.at[pl.ds(r0, rpt_last)],
                      accum.at[pl.ds(r0, rpt_last)])

    @pl.when(jnp.logical_not(is_last))
    def _():
      pltpu.sync_copy(zrow_hbm.at[pl.ds(r0, rpt0)],
                      accum.at[pl.ds(r0, rpt0)])

    if with_counts:
      pltpu.sync_copy(zcnt_hbm, cntv)
    plsc.subcore_barrier()

    base = wid * e_per_tile
    ones = jnp.ones((L,), jnp.float32)

    def issue_idx(c, p):
      off = base + c * K
      pltpu.async_copy(edges_hbm.at[pl.ds(off, K)], srcs[p], semi[p])
      pltpu.async_copy(edges_hbm.at[pl.ds(E + off, K)], dsts[p], semi[p])

    def wait_idx(p):
      pltpu.make_async_copy(edges_hbm.at[pl.ds(0, K)], srcs[p], semi[p]).wait()
      pltpu.make_async_copy(edges_hbm.at[pl.ds(0, K)], dsts[p], semi[p]).wait()

    def issue_gather(p):
      h = K // 2
      pltpu.async_copy(x_hbm.at[srcs[p].at[pl.ds(0, h)]],
                       rows[p].at[pl.ds(0, h)], semg[p])
      pltpu.async_copy(x_hbm.at[srcs[p].at[pl.ds(h, h)]],
                       rows[p].at[pl.ds(h, h)], semg[p])

    def wait_gather(p):
      pltpu.make_async_copy(x_hbm.at[srcs[p]], rows[p], semg[p]).wait()

    def do_counts(dref, n):
      if with_counts:
        for t in range(n // L):
          d16 = dref[pl.ds(t * L, L)]
          plsc.addupdate_scatter(cntv, [d16], ones)

    def scatter(p):
      pltpu.sync_copy(rows[p], accum.at[dsts[p]], add=True)

    issue_idx(0, 0)
    issue_idx(1, 1)
    wait_idx(0)
    issue_gather(0)

    def pair(j, _):
      for b in range(2):
        c = 2 * j + b
        p, q = b, 1 - b
        wait_idx(q)
        issue_gather(q)
        do_counts(dsts[p], K)
        wait_gather(p)
        scatter(p)
        issue_idx(c + 2, p)
      return 0

    lax.fori_loop(0, (n_full - 2) // 2, pair, 0)

    wait_idx(1)
    issue_gather(1)
    do_counts(dsts[0], K)
    wait_gather(0)
    scatter(0)
    do_counts(dsts[1], K)
    wait_gather(1)
    scatter(1)

    if rem:
      off = base + n_full * K
      pltpu.sync_copy(edges_hbm.at[pl.ds(off, rem)], srcv_t)
      pltpu.sync_copy(edges_hbm.at[pl.ds(E + off, rem)], dstv_t)
      pltpu.async_copy(x_hbm.at[srcv_t], rows_t, sem_t).wait()
      pltpu.sync_copy(rows_t, accum.at[dstv_t], add=True)
      do_counts(dstv_t, rem)

    plsc.subcore_barrier()

    @pl.when(is_last)
    def _():
      pltpu.sync_copy(accum.at[pl.ds(r0, rpt_last)],
                      out_p.at[cid, pl.ds(r0, rpt_last)])

    @pl.when(jnp.logical_not(is_last))
    def _():
      pltpu.sync_copy(accum.at[pl.ds(r0, rpt0)],
                      out_p.at[cid, pl.ds(r0, rpt0)])

    if with_counts:
      pltpu.sync_copy(cntv, out_cnt.at[pl.ds(wid * N, N)])

  return pl.kernel(
      body, out_type=out_type, mesh=mesh, scratch_types=scratch,
      compiler_params=pltpu.CompilerParams(needs_layout_passes=False))


def _tc_layer_kernel(p_ref, cnt_ref, x_ref, wlt_ref, bl_ref, wrt_ref, o_ref,
                     *, relu):
  s = p_ref[0] + p_ref[1]
  cnt = jnp.sum(cnt_ref[...], axis=1, keepdims=True)
  mean = s / jnp.maximum(cnt, 1.0)
  out = (jax.lax.dot(mean, wlt_ref[...],
                     preferred_element_type=jnp.float32,
                     precision=jax.lax.Precision.HIGHEST)
         + bl_ref[...]
         + jax.lax.dot(x_ref[...], wrt_ref[...],
                       preferred_element_type=jnp.float32,
                       precision=jax.lax.Precision.HIGHEST))
  nrm = jnp.sqrt(jnp.sum(out * out, axis=-1, keepdims=True))
  out = out / jnp.maximum(nrm, 1e-12)
  if relu:
    out = jnp.maximum(out, 0.0)
  o_ref[...] = out


def _tc_layer(p, counts, xin, wlt, bl2d, wrt, relu):
  N, D = xin.shape
  NW = counts.shape[1]
  BN = 1000
  assert N % BN == 0
  return pl.pallas_call(
      functools.partial(_tc_layer_kernel, relu=relu),
      grid=(N // BN,),
      in_specs=[
          pl.BlockSpec((NC, BN, D), lambda i: (0, i, 0)),
          pl.BlockSpec((BN, NW), lambda i: (i, 0)),
          pl.BlockSpec((BN, D), lambda i: (i, 0)),
          pl.BlockSpec((D, D), lambda i: (0, 0)),
          pl.BlockSpec((1, D), lambda i: (0, 0)),
          pl.BlockSpec((D, D), lambda i: (0, 0)),
      ],
      out_specs=pl.BlockSpec((BN, D), lambda i: (i, 0)),
      out_shape=jax.ShapeDtypeStruct((N, D), jnp.float32),
  )(p, counts, xin, wlt, bl2d, wrt)


@jax.jit
def kernel(x, edge_index, Wl1, bl1, Wr1, Wl2, bl2, Wr2):
  N, D = x.shape
  E = edge_index.shape[1]
  edges = edge_index.reshape(2 * E)
  zrow = jnp.zeros((N, D), jnp.float32)
  zcnt = jnp.zeros((N,), jnp.float32)

  agg1 = _sc_aggregate(N, D, E, with_counts=True)
  p1, counts = agg1(x, edges, zrow, zcnt)
  counts_t = counts.reshape(NC * NS, N).T
  h = _tc_layer(p1, counts_t, x, Wl1.T, bl1.reshape(1, D), Wr1.T, relu=True)

  agg2 = _sc_aggregate(N, D, E, with_counts=False)
  (p2,) = agg2(h, edges, zrow, zcnt)
  return _tc_layer(p2, counts_t, h, Wl2.T, bl2.reshape(1, D), Wr2.T, relu=False)


# speedup vs baseline: 1.0547x; 1.0547x over previous
import functools

import jax
import jax.numpy as jnp
from jax import lax
from jax.experimental import pallas as pl
from jax.experimental.pallas import tpu as pltpu
from jax.experimental.pallas import tpu_sc as plsc

NC = 2
NS = 16
L = 16
K = 128


def _sc_aggregate(N, D, E, with_counts):
  NW = NC * NS
  assert E % NW == 0
  e_per_tile = E // NW
  n_full = e_per_tile // K
  rem = e_per_tile - n_full * K
  assert rem % 8 == 0
  rpt0 = (N // NS) // 8 * 8
  rpt_last = N - (NS - 1) * rpt0

  mesh = plsc.VectorSubcoreMesh(core_axis_name="c", subcore_axis_name="s")

  assert n_full >= 4 and n_full % 2 == 0

  out_type = [jax.ShapeDtypeStruct((NC, N, D), jnp.float32)]
  scratch = (
      [pltpu.VMEM((K,), jnp.int32)] * 2 +
      [pltpu.VMEM((K,), jnp.int32)] * 2 +
      [pltpu.VMEM((K, D), jnp.float32)] * 2 +
      [pltpu.VMEM((rem if rem else 8,), jnp.int32)] * 2 +
      [pltpu.VMEM((rem if rem else 8, D), jnp.float32)] +
      [pltpu.VMEM_SHARED((N, D), jnp.float32)] +
      [pltpu.SemaphoreType.DMA] * 2 +
      [pltpu.SemaphoreType.DMA] * 2 +
      [pltpu.SemaphoreType.DMA]
  )
  if with_counts:
    out_type.append(jax.ShapeDtypeStruct((NW * N,), jnp.float32))
    scratch.append(pltpu.VMEM((N,), jnp.float32))

  def body(x_hbm, edges_hbm, zrow_hbm, zcnt_hbm, *rest):
    if with_counts:
      out_p, out_cnt = rest[0], rest[1]
      rest = rest[2:]
    else:
      out_p = rest[0]
      rest = rest[1:]
    srcs, dsts, rows = rest[0:2], rest[2:4], rest[4:6]
    srcv_t, dstv_t, rows_t, accum = rest[6:10]
    semi, semg = rest[10:12], rest[12:14]
    sem_t = rest[14]
    cntv = rest[15] if with_counts else None

    cid = lax.axis_index("c")
    sid = lax.axis_index("s")
    wid = sid * NC + cid

    r0 = sid * rpt0
    is_last = sid == NS - 1

    @pl.when(is_last)
    def _():
      pltpu.sync_copy(zrow_hbm.at[pl.ds(r0, rpt_last)],
                      accum.at[pl.ds(r0, rpt_last)])

    @pl.when(jnp.logical_not(is_last))
    def _():
      pltpu.sync_copy(zrow_hbm.at[pl.ds(r0, rpt0)],
                      accum.at[pl.ds(r0, rpt0)])

    if with_counts:
      pltpu.sync_copy(zcnt_hbm, cntv)
    plsc.subcore_barrier()

    base = wid * e_per_tile
    ones = jnp.ones((L,), jnp.float32)

    def issue_idx(c, p):
      off = base + c * K
      pltpu.async_copy(edges_hbm.at[pl.ds(off, K)], srcs[p], semi[p])
      pltpu.async_copy(edges_hbm.at[pl.ds(E + off, K)], dsts[p], semi[p])

    def wait_idx(p):
      pltpu.make_async_copy(edges_hbm.at[pl.ds(0, K)], srcs[p], semi[p]).wait()
      pltpu.make_async_copy(edges_hbm.at[pl.ds(0, K)], dsts[p], semi[p]).wait()

    def issue_gather(p):
      h = K // 2
      pltpu.async_copy(x_hbm.at[srcs[p].at[pl.ds(0, h)]],
                       rows[p].at[pl.ds(0, h)], semg[p])
      pltpu.async_copy(x_hbm.at[srcs[p].at[pl.ds(h, h)]],
                       rows[p].at[pl.ds(h, h)], semg[p])

    def wait_gather(p):
      pltpu.make_async_copy(x_hbm.at[srcs[p]], rows[p], semg[p]).wait()

    def do_counts(dref, n):
      if with_counts:
        for t in range(n // L):
          d16 = dref[pl.ds(t * L, L)]
          plsc.addupdate_scatter(cntv, [d16], ones)

    def scatter(p):
      pltpu.sync_copy(rows[p], accum.at[dsts[p]], add=True)

    issue_idx(0, 0)
    issue_idx(1, 1)
    wait_idx(0)
    issue_gather(0)

    def pair(j, _):
      for b in range(2):
        c = 2 * j + b
        p, q = b, 1 - b
        wait_idx(q)
        issue_gather(q)
        do_counts(dsts[p], K)
        wait_gather(p)
        scatter(p)
        issue_idx(c + 2, p)
      return 0

    lax.fori_loop(0, (n_full - 2) // 2, pair, 0)

    wait_idx(1)
    issue_gather(1)
    do_counts(dsts[0], K)
    wait_gather(0)
    scatter(0)
    do_counts(dsts[1], K)
    wait_gather(1)
    scatter(1)

    if rem:
      off = base + n_full * K
      pltpu.sync_copy(edges_hbm.at[pl.ds(off, rem)], srcv_t)
      pltpu.sync_copy(edges_hbm.at[pl.ds(E + off, rem)], dstv_t)
      pltpu.async_copy(x_hbm.at[srcv_t], rows_t, sem_t).wait()
      pltpu.sync_copy(rows_t, accum.at[dstv_t], add=True)
      do_counts(dstv_t, rem)

    plsc.subcore_barrier()

    @pl.when(is_last)
    def _():
      pltpu.sync_copy(accum.at[pl.ds(r0, rpt_last)],
                      out_p.at[cid, pl.ds(r0, rpt_last)])

    @pl.when(jnp.logical_not(is_last))
    def _():
      pltpu.sync_copy(accum.at[pl.ds(r0, rpt0)],
                      out_p.at[cid, pl.ds(r0, rpt0)])

    if with_counts:
      pltpu.sync_copy(cntv, out_cnt.at[pl.ds(wid * N, N)])

  return pl.kernel(
      body, out_type=out_type, mesh=mesh, scratch_types=scratch,
      compiler_params=pltpu.CompilerParams(needs_layout_passes=False))


def _tc_layer_kernel(p_ref, cnt_ref, x_ref, wlt_ref, bl_ref, wrt_ref, o_ref,
                     *, relu):
  s = p_ref[0] + p_ref[1]
  cnt = jnp.sum(cnt_ref[...], axis=1, keepdims=True)
  mean = s / jnp.maximum(cnt, 1.0)
  out = (jax.lax.dot(mean, wlt_ref[...], preferred_element_type=jnp.float32)
         + bl_ref[...]
         + jax.lax.dot(x_ref[...], wrt_ref[...], preferred_element_type=jnp.float32))
  nrm = jnp.sqrt(jnp.sum(out * out, axis=-1, keepdims=True))
  out = out / jnp.maximum(nrm, 1e-12)
  if relu:
    out = jnp.maximum(out, 0.0)
  o_ref[...] = out


def _tc_layer(p, counts, xin, wlt, bl2d, wrt, relu):
  N, D = xin.shape
  NW = counts.shape[1]
  BN = 1000
  assert N % BN == 0
  return pl.pallas_call(
      functools.partial(_tc_layer_kernel, relu=relu),
      grid=(N // BN,),
      in_specs=[
          pl.BlockSpec((NC, BN, D), lambda i: (0, i, 0)),
          pl.BlockSpec((BN, NW), lambda i: (i, 0)),
          pl.BlockSpec((BN, D), lambda i: (i, 0)),
          pl.BlockSpec((D, D), lambda i: (0, 0)),
          pl.BlockSpec((1, D), lambda i: (0, 0)),
          pl.BlockSpec((D, D), lambda i: (0, 0)),
      ],
      out_specs=pl.BlockSpec((BN, D), lambda i: (i, 0)),
      out_shape=jax.ShapeDtypeStruct((N, D), jnp.float32),
  )(p, counts, xin, wlt, bl2d, wrt)


@jax.jit
def kernel(x, edge_index, Wl1, bl1, Wr1, Wl2, bl2, Wr2):
  N, D = x.shape
  E = edge_index.shape[1]
  edges = edge_index.reshape(2 * E)
  zrow = jnp.zeros((N, D), jnp.float32)
  zcnt = jnp.zeros((N,), jnp.float32)

  agg1 = _sc_aggregate(N, D, E, with_counts=True)
  p1, counts = agg1(x, edges, zrow, zcnt)
  counts_t = counts.reshape(NC * NS, N).T
  h = _tc_layer(p1, counts_t, x, Wl1.T, bl1.reshape(1, D), Wr1.T, relu=True)

  agg2 = _sc_aggregate(N, D, E, with_counts=False)
  (p2,) = agg2(h, edges, zrow, zcnt)
  return _tc_layer(p2, counts_t, h, Wl2.T, bl2.reshape(1, D), Wr2.T, relu=False)
